# baseline (speedup 1.0000x reference)
import functools

import jax
import jax.numpy as jnp
from jax import lax
from jax.experimental import pallas as pl
from jax.experimental.pallas import tpu as pltpu
from jax.experimental.pallas import tpu_sc as plsc

BATCH = 16384
EMB = 32
HID = 16
PACK = 4
LANES = 128

NUM_CORES = 2
NUM_SUBCORES = 16
NUM_WORKERS = NUM_CORES * NUM_SUBCORES
ROWS_PER_WORKER = BATCH // NUM_WORKERS
CHUNK = 128
N_CHUNKS = ROWS_PER_WORKER // CHUNK
GROUPS = CHUNK // 16


def _sc_gather_mul(x0_hbm, x1_hbm, ptab_hbm, itab_hbm, ht_hbm,
                   x0_v, x1_v, idx0_v, idx1_v, prows_v, irows_v, ht_v,
                   sem0, sem1):
  wid = lax.axis_index("s") * NUM_CORES + lax.axis_index("c")
  base = wid * ROWS_PER_WORKER

  pltpu.sync_copy(x0_hbm.at[pl.ds(wid * N_CHUNKS, N_CHUNKS)], x0_v)
  pltpu.sync_copy(x1_hbm.at[pl.ds(wid * N_CHUNKS, N_CHUNKS)], x1_v)

  for j in range(N_CHUNKS):
    for g in range(GROUPS):
      sl = pl.ds(g * 16, 16)
      idx0_v[j, sl] = lax.shift_right_logical(x0_v[j, sl], 2)
      idx1_v[j, sl] = lax.shift_right_logical(x1_v[j, sl], 2)

  lane = lax.iota(jnp.int32, 16)

  def do_chunk(j, buf):
    cp = pltpu.make_async_copy(
        ptab_hbm.at[idx0_v.at[j]], prows_v.at[buf], sem0)
    ci = pltpu.make_async_copy(
        itab_hbm.at[idx1_v.at[j]], irows_v.at[buf], sem1)
    cp.start()
    ci.start()
    cp.wait()
    ci.wait()
    for g in range(GROUPS):
      sl = pl.ds(g * 16, 16)
      row = lane + (g * 16)
      off0 = (x0_v[j, sl] & 3) * 32
      off1 = (x1_v[j, sl] & 3) * 32
      col = j * CHUNK + g * 16
      for f in range(EMB):
        pv = plsc.load_gather(prows_v.at[buf], [row, off0 + f])
        iv = plsc.load_gather(irows_v.at[buf], [row, off1 + f])
        ht_v[f, pl.ds(col, 16)] = pv * iv
    return buf

  lax.fori_loop(0, N_CHUNKS, lambda j, c: do_chunk(j, 0), 0)

  pltpu.sync_copy(ht_v, ht_hbm.at[:, pl.ds(base, ROWS_PER_WORKER)])


def _tc_bn_mlp(ht_ref, gamma_ref, beta_ref, w1_ref, b1_ref, w2_ref, b2_ref,
               out_ref):
  h = ht_ref[...]
  s = jnp.sum(h, axis=1, keepdims=True)
  sq = jnp.sum(h * h, axis=1, keepdims=True)
  mean = s * (1.0 / BATCH)
  var = sq * (1.0 / BATCH) - mean * mean
  a = gamma_ref[...] * lax.rsqrt(var + 1e-5)
  c = beta_ref[...] - mean * a
  hn = h * a + c

  z1 = jnp.dot(w1_ref[...], hn, preferred_element_type=jnp.float32)
  z1 = z1 + b1_ref[...]
  scale = 1.0507009873554804934193349852946
  alpha = 1.6732632423543772848170429916717
  z1 = scale * jnp.where(z1 > 0, z1, alpha * (jnp.exp(z1) - 1.0))
  z2 = jnp.dot(w2_ref[...], z1, preferred_element_type=jnp.float32)
  z2 = z2 + b2_ref[...]
  out_ref[...] = jax.nn.sigmoid(z2)


@jax.jit
def kernel(x, playlist_emb, item_emb, bn_gamma, bn_beta, W1, b1, W2, b2):
  xt = x.T.astype(jnp.int32)
  x0 = xt[0].reshape(LANES, LANES)
  x1 = xt[1].reshape(LANES, LANES)
  ptab = playlist_emb.reshape(-1, LANES)
  itab = item_emb.reshape(-1, LANES)

  mesh = plsc.VectorSubcoreMesh(core_axis_name="c", subcore_axis_name="s")
  gather_mul = pl.kernel(
      _sc_gather_mul,
      out_type=jax.ShapeDtypeStruct((EMB, BATCH), jnp.float32),
      mesh=mesh,
      compiler_params=pltpu.CompilerParams(
          use_tc_tiling_on_sc=True, needs_layout_passes=False),
      scratch_types=[
          pltpu.VMEM((N_CHUNKS, CHUNK), jnp.int32),
          pltpu.VMEM((N_CHUNKS, CHUNK), jnp.int32),
          pltpu.VMEM((N_CHUNKS, CHUNK), jnp.int32),
          pltpu.VMEM((N_CHUNKS, CHUNK), jnp.int32),
          pltpu.VMEM((1, CHUNK, LANES), jnp.float32),
          pltpu.VMEM((1, CHUNK, LANES), jnp.float32),
          pltpu.VMEM((EMB, ROWS_PER_WORKER), jnp.float32),
          pltpu.SemaphoreType.DMA,
          pltpu.SemaphoreType.DMA,
      ],
  )
  ht = gather_mul(x0, x1, ptab, itab)

  out = pl.pallas_call(
      _tc_bn_mlp,
      out_shape=jax.ShapeDtypeStruct((1, BATCH), jnp.float32),
  )(ht, bn_gamma.reshape(EMB, 1), bn_beta.reshape(EMB, 1),
    W1, b1.reshape(HID, 1), W2, b2.reshape(1, 1))

  return out.reshape(BATCH, 1)


# speedup vs baseline: 1.1477x; 1.1477x over previous
import functools

import jax
import jax.numpy as jnp
from jax import lax
from jax.experimental import pallas as pl
from jax.experimental.pallas import tpu as pltpu
from jax.experimental.pallas import tpu_sc as plsc

BATCH = 16384
EMB = 32
HID = 16
PACK = 4
LANES = 128

NUM_CORES = 2
NUM_SUBCORES = 16
NUM_WORKERS = NUM_CORES * NUM_SUBCORES
ROWS_PER_WORKER = BATCH // NUM_WORKERS
CHUNK = 128
N_CHUNKS = ROWS_PER_WORKER // CHUNK
GROUPS = CHUNK // 16
NBUF = 2


def _sc_gather_mul(x0_hbm, x1_hbm, ptab_hbm, itab_hbm, h_hbm,
                   x0_v, x1_v, idx0_v, idx1_v, off_v,
                   prows_v, irows_v, h_v, sem0, sem1):
  wid = lax.axis_index("s") * NUM_CORES + lax.axis_index("c")
  base = wid * ROWS_PER_WORKER

  pltpu.sync_copy(x0_hbm.at[pl.ds(wid * N_CHUNKS, N_CHUNKS)], x0_v)
  pltpu.sync_copy(x1_hbm.at[pl.ds(wid * N_CHUNKS, N_CHUNKS)], x1_v)

  for j in range(N_CHUNKS):
    for g in range(GROUPS):
      sl = pl.ds(g * 16, 16)
      idx0_v[j, sl] = lax.shift_right_logical(x0_v[j, sl], 2)
      idx1_v[j, sl] = lax.shift_right_logical(x1_v[j, sl], 2)
      off_v[0, j, sl] = (x0_v[j, sl] & 3) * EMB
      off_v[1, j, sl] = (x1_v[j, sl] & 3) * EMB

  def fire(j, buf):
    cp = pltpu.make_async_copy(
        ptab_hbm.at[idx0_v.at[j]], prows_v.at[buf], sem0)
    ci = pltpu.make_async_copy(
        itab_hbm.at[idx1_v.at[j]], irows_v.at[buf], sem1)
    cp.start()
    ci.start()
    return cp, ci

  def drain(buf):
    pltpu.make_async_copy(
        ptab_hbm.at[idx0_v.at[0]], prows_v.at[buf], sem0).wait()
    pltpu.make_async_copy(
        itab_hbm.at[idx1_v.at[0]], irows_v.at[buf], sem1).wait()

  fire(0, 0)

  lane = lax.iota(jnp.int32, 16)

  def extract(j, buf):
    def body(g, _):
      off0g = off_v[0, j, pl.ds(g * 16, 16)]
      off1g = off_v[1, j, pl.ds(g * 16, 16)]
      for rr in range(16):
        sel = jnp.full((16,), rr, jnp.int32)
        col0 = off0g[sel] + lane
        col1 = off1g[sel] + lane
        row = jnp.full((16,), g * 16 + rr, jnp.int32)
        p_lo = plsc.load_gather(prows_v.at[buf], [row, col0])
        p_hi = plsc.load_gather(prows_v.at[buf], [row, col0 + 16])
        i_lo = plsc.load_gather(irows_v.at[buf], [row, col1])
        i_hi = plsc.load_gather(irows_v.at[buf], [row, col1 + 16])
        q = j * (CHUNK // PACK) + g * PACK + rr // PACK
        d = (rr % PACK) * EMB
        h_v[q, pl.ds(d, 16)] = p_lo * i_lo
        h_v[q, pl.ds(d + 16, 16)] = p_hi * i_hi
      return _
    lax.fori_loop(0, GROUPS, body, 0)

  for j in range(N_CHUNKS):
    if j + 1 < N_CHUNKS:
      fire(j + 1, (j + 1) % NBUF)
    drain(j % NBUF)
    extract(j, j % NBUF)

  pltpu.sync_copy(h_v, h_hbm.at[pl.ds(wid * (ROWS_PER_WORKER // PACK),
                                      ROWS_PER_WORKER // PACK)])


def _tc_bn_mlp(h_ref, gamma_ref, beta_ref, w1_ref, b1_ref, w2_ref, b2_ref,
               out_ref):
  h = h_ref[...]
  s = jnp.sum(h, axis=0, keepdims=True)
  sq = jnp.sum(h * h, axis=0, keepdims=True)
  s32 = (s[:, 0:32] + s[:, 32:64]) + (s[:, 64:96] + s[:, 96:128])
  sq32 = (sq[:, 0:32] + sq[:, 32:64]) + (sq[:, 64:96] + sq[:, 96:128])
  mean = s32 * (1.0 / BATCH)
  var = sq32 * (1.0 / BATCH) - mean * mean
  a32 = gamma_ref[...] * lax.rsqrt(var + 1e-5)
  c32 = beta_ref[...] - mean * a32
  a = jnp.concatenate([a32, a32, a32, a32], axis=1)
  c = jnp.concatenate([c32, c32, c32, c32], axis=1)
  hn = h * a + c

  z1 = jnp.dot(hn, w1_ref[...], preferred_element_type=jnp.float32)
  z1 = z1 + b1_ref[...]
  scale = 1.0507009873554804934193349852946
  alpha = 1.6732632423543772848170429916717
  z1 = scale * jnp.where(z1 > 0, z1, alpha * (jnp.exp(z1) - 1.0))
  z2 = jnp.dot(z1, w2_ref[...], preferred_element_type=jnp.float32)
  z2 = z2 + b2_ref[...]
  out_ref[...] = jax.nn.sigmoid(z2)


@jax.jit
def kernel(x, playlist_emb, item_emb, bn_gamma, bn_beta, W1, b1, W2, b2):
  xt = x.T.astype(jnp.int32)
  x0 = xt[0].reshape(LANES, LANES)
  x1 = xt[1].reshape(LANES, LANES)
  ptab = playlist_emb.reshape(-1, LANES)
  itab = item_emb.reshape(-1, LANES)

  mesh = plsc.VectorSubcoreMesh(core_axis_name="c", subcore_axis_name="s")
  gather_mul = pl.kernel(
      _sc_gather_mul,
      out_type=jax.ShapeDtypeStruct((BATCH // PACK, PACK * EMB), jnp.float32),
      mesh=mesh,
      compiler_params=pltpu.CompilerParams(
          use_tc_tiling_on_sc=True, needs_layout_passes=False),
      scratch_types=[
          pltpu.VMEM((N_CHUNKS, CHUNK), jnp.int32),
          pltpu.VMEM((N_CHUNKS, CHUNK), jnp.int32),
          pltpu.VMEM((N_CHUNKS, CHUNK), jnp.int32),
          pltpu.VMEM((N_CHUNKS, CHUNK), jnp.int32),
          pltpu.VMEM((2, N_CHUNKS, CHUNK), jnp.int32),
          pltpu.VMEM((NBUF, CHUNK, LANES), jnp.float32),
          pltpu.VMEM((NBUF, CHUNK, LANES), jnp.float32),
          pltpu.VMEM((ROWS_PER_WORKER // PACK, PACK * EMB), jnp.float32),
          pltpu.SemaphoreType.DMA,
          pltpu.SemaphoreType.DMA,
      ],
  )
  h2 = gather_mul(x0, x1, ptab, itab)

  eye = jnp.eye(PACK, dtype=jnp.float32)
  w1big = jnp.kron(eye, W1.T)
  b1big = jnp.tile(b1, PACK).reshape(1, PACK * HID)
  w2big = jnp.kron(eye, W2.T)
  b2big = jnp.tile(b2, PACK).reshape(1, PACK)

  out = pl.pallas_call(
      _tc_bn_mlp,
      out_shape=jax.ShapeDtypeStruct((BATCH // PACK, PACK), jnp.float32),
  )(h2, bn_gamma.reshape(1, EMB), bn_beta.reshape(1, EMB),
    w1big, b1big, w2big, b2big)

  return out.reshape(BATCH, 1)


# speedup vs baseline: 1.7648x; 1.5377x over previous
import functools

import jax
import jax.numpy as jnp
from jax import lax
from jax.experimental import pallas as pl
from jax.experimental.pallas import tpu as pltpu
from jax.experimental.pallas import tpu_sc as plsc

BATCH = 16384
EMB = 32
HID = 16
PACK = 4
LANES = 128

NUM_CORES = 2
NUM_SUBCORES = 16
NUM_WORKERS = NUM_CORES * NUM_SUBCORES
ROWS_PER_WORKER = BATCH // NUM_WORKERS
CHUNK = 128
N_CHUNKS = ROWS_PER_WORKER // CHUNK
GROUPS = CHUNK // 16
NBUF = 2


def _sc_gather_mul(x0_hbm, x1_hbm, ptab_hbm, itab_hbm, h_hbm,
                   x0_v, x1_v, idx0_v, idx1_v, off_v,
                   prows_v, irows_v, h_v, sem0, sem1):
  wid = lax.axis_index("s") * NUM_CORES + lax.axis_index("c")
  base = wid * ROWS_PER_WORKER

  pltpu.sync_copy(x0_hbm.at[pl.ds(wid * N_CHUNKS, N_CHUNKS)], x0_v)
  pltpu.sync_copy(x1_hbm.at[pl.ds(wid * N_CHUNKS, N_CHUNKS)], x1_v)

  for j in range(N_CHUNKS):
    for g in range(GROUPS):
      sl = pl.ds(g * 16, 16)
      idx0_v[j, sl] = lax.shift_right_logical(x0_v[j, sl], 2)
      idx1_v[j, sl] = lax.shift_right_logical(x1_v[j, sl], 2)
      off_v[0, j, sl] = (x0_v[j, sl] & 3) * EMB
      off_v[1, j, sl] = (x1_v[j, sl] & 3) * EMB

  def fire(j, buf):
    cp = pltpu.make_async_copy(
        ptab_hbm.at[idx0_v.at[j]], prows_v.at[buf], sem0)
    ci = pltpu.make_async_copy(
        itab_hbm.at[idx1_v.at[j]], irows_v.at[buf], sem1)
    cp.start()
    ci.start()
    return cp, ci

  def drain(buf):
    pltpu.make_async_copy(
        ptab_hbm.at[idx0_v.at[0]], prows_v.at[buf], sem0).wait()
    pltpu.make_async_copy(
        itab_hbm.at[idx1_v.at[0]], irows_v.at[buf], sem1).wait()

  fire(0, 0)

  lane = lax.iota(jnp.int32, 16)

  def extract(j, buf):
    def body(g, _):
      off0g = off_v[0, j, pl.ds(g * 16, 16)]
      off1g = off_v[1, j, pl.ds(g * 16, 16)]
      for rr in range(16):
        sel = jnp.full((16,), rr, jnp.int32)
        col0 = off0g[sel] + lane
        col1 = off1g[sel] + lane
        row = jnp.full((16,), g * 16 + rr, jnp.int32)
        p_lo = plsc.load_gather(prows_v.at[buf], [row, col0])
        p_hi = plsc.load_gather(prows_v.at[buf], [row, col0 + 16])
        i_lo = plsc.load_gather(irows_v.at[buf], [row, col1])
        i_hi = plsc.load_gather(irows_v.at[buf], [row, col1 + 16])
        q = j * (CHUNK // PACK) + g * PACK + rr // PACK
        d = (rr % PACK) * EMB
        h_v[q, pl.ds(d, 16)] = p_lo * i_lo
        h_v[q, pl.ds(d + 16, 16)] = p_hi * i_hi
      return _
    lax.fori_loop(0, GROUPS, body, 0)

  for j in range(N_CHUNKS):
    if j + 1 < N_CHUNKS:
      fire(j + 1, (j + 1) % NBUF)
    drain(j % NBUF)
    extract(j, j % NBUF)

  pltpu.sync_copy(h_v, h_hbm.at[pl.ds(wid * (ROWS_PER_WORKER // PACK),
                                      ROWS_PER_WORKER // PACK)])


def _tc_bn_mlp(h_ref, gamma_ref, beta_ref, w1_ref, b1_ref, w2_ref, b2_ref,
               out_ref):
  h = h_ref[...]
  s = jnp.sum(h, axis=0, keepdims=True)
  sq = jnp.sum(h * h, axis=0, keepdims=True)
  s32 = (s[:, 0:32] + s[:, 32:64]) + (s[:, 64:96] + s[:, 96:128])
  sq32 = (sq[:, 0:32] + sq[:, 32:64]) + (sq[:, 64:96] + sq[:, 96:128])
  mean = s32 * (1.0 / BATCH)
  var = sq32 * (1.0 / BATCH) - mean * mean
  a32 = gamma_ref[...] * lax.rsqrt(var + 1e-5)
  c32 = beta_ref[...] - mean * a32
  a = jnp.concatenate([a32, a32, a32, a32], axis=1)
  c = jnp.concatenate([c32, c32, c32, c32], axis=1)
  hn = h * a + c

  z1 = jnp.dot(hn, w1_ref[...], preferred_element_type=jnp.float32)
  z1 = z1 + b1_ref[...]
  scale = 1.0507009873554804934193349852946
  alpha = 1.6732632423543772848170429916717
  z1 = scale * jnp.where(z1 > 0, z1, alpha * (jnp.exp(z1) - 1.0))
  z2 = jnp.dot(z1, w2_ref[...], preferred_element_type=jnp.float32)
  z2 = z2 + b2_ref[...]
  out_ref[...] = jax.nn.sigmoid(z2)


USED_ROWS = 40000
PAD_COLS = 40064
PACKED_ROWS = PAD_COLS // PACK
PCOLS = 1280
PTAIL = PAD_COLS - 31 * PCOLS


def _sc_pack(pt_hbm, it_hbm, ptab_hbm, itab_hbm, in_v, out_v):
  wid = lax.axis_index("s") * NUM_CORES + lax.axis_index("c")
  lane = lax.iota(jnp.int32, 16)
  rots = [(lane + k) & 15 for k in range(16)]

  def pack_block(src_hbm, dst_hbm, c0, cols):
    c0 = pl.multiple_of(c0, 128)
    pltpu.sync_copy(src_hbm.at[:, pl.ds(c0, cols)],
                    in_v.at[:, pl.ds(0, cols)])

    def body(cb, _):
      c_base = cb * 16
      for f_base in range(0, EMB, 16):
        fl = lane + f_base
        for k in range(16):
          col = c_base + rots[k]
          v = plsc.load_gather(in_v, [fl, col])
          flat = ((col << 5) + f_base) + lane
          plsc.store_scatter(out_v, [lax.shift_right_logical(flat, 7),
                                     flat & 127], v)
      return _

    lax.fori_loop(0, cols // 16, body, 0)
    q0 = pl.multiple_of(c0 // PACK, 32)
    pltpu.sync_copy(out_v.at[pl.ds(0, cols // PACK)],
                    dst_hbm.at[pl.ds(q0, cols // PACK)])

  @pl.when(wid < 31)
  def _():
    pack_block(pt_hbm, ptab_hbm, wid * PCOLS, PCOLS)
    pack_block(it_hbm, itab_hbm, wid * PCOLS, PCOLS)

  @pl.when(wid == 31)
  def _():
    pack_block(pt_hbm, ptab_hbm, 31 * PCOLS, PTAIL)
    pack_block(it_hbm, itab_hbm, 31 * PCOLS, PTAIL)


@jax.jit
def kernel(x, playlist_emb, item_emb, bn_gamma, bn_beta, W1, b1, W2, b2):
  xt = x.T.astype(jnp.int32)
  x0 = xt[0].reshape(LANES, LANES)
  x1 = xt[1].reshape(LANES, LANES)

  mesh = plsc.VectorSubcoreMesh(core_axis_name="c", subcore_axis_name="s")
  pack = pl.kernel(
      _sc_pack,
      out_type=(jax.ShapeDtypeStruct((PACKED_ROWS, LANES), jnp.float32),
                jax.ShapeDtypeStruct((PACKED_ROWS, LANES), jnp.float32)),
      mesh=mesh,
      compiler_params=pltpu.CompilerParams(
          use_tc_tiling_on_sc=True, needs_layout_passes=False),
      scratch_types=[
          pltpu.VMEM((EMB, PCOLS), jnp.float32),
          pltpu.VMEM((PCOLS // PACK, LANES), jnp.float32),
      ],
  )
  ptp = jnp.pad(playlist_emb.T[:, :USED_ROWS], ((0, 0), (0, PAD_COLS - USED_ROWS)))
  itp = jnp.pad(item_emb.T, ((0, 0), (0, PAD_COLS - USED_ROWS)))
  ptab, itab = pack(ptp, itp)

  mesh = plsc.VectorSubcoreMesh(core_axis_name="c", subcore_axis_name="s")
  gather_mul = pl.kernel(
      _sc_gather_mul,
      out_type=jax.ShapeDtypeStruct((BATCH // PACK, PACK * EMB), jnp.float32),
      mesh=mesh,
      compiler_params=pltpu.CompilerParams(
          use_tc_tiling_on_sc=True, needs_layout_passes=False),
      scratch_types=[
          pltpu.VMEM((N_CHUNKS, CHUNK), jnp.int32),
          pltpu.VMEM((N_CHUNKS, CHUNK), jnp.int32),
          pltpu.VMEM((N_CHUNKS, CHUNK), jnp.int32),
          pltpu.VMEM((N_CHUNKS, CHUNK), jnp.int32),
          pltpu.VMEM((2, N_CHUNKS, CHUNK), jnp.int32),
          pltpu.VMEM((NBUF, CHUNK, LANES), jnp.float32),
          pltpu.VMEM((NBUF, CHUNK, LANES), jnp.float32),
          pltpu.VMEM((ROWS_PER_WORKER // PACK, PACK * EMB), jnp.float32),
          pltpu.SemaphoreType.DMA,
          pltpu.SemaphoreType.DMA,
      ],
  )
  h2 = gather_mul(x0, x1, ptab, itab)

  eye = jnp.eye(PACK, dtype=jnp.float32)
  w1big = jnp.kron(eye, W1.T)
  b1big = jnp.tile(b1, PACK).reshape(1, PACK * HID)
  w2big = jnp.kron(eye, W2.T)
  b2big = jnp.tile(b2, PACK).reshape(1, PACK)

  out = pl.pallas_call(
      _tc_bn_mlp,
      out_shape=jax.ShapeDtypeStruct((BATCH // PACK, PACK), jnp.float32),
  )(h2, bn_gamma.reshape(1, EMB), bn_beta.reshape(1, EMB),
    w1big, b1big, w2big, b2big)

  return out.reshape(BATCH, 1)


# speedup vs baseline: 1.8547x; 1.0509x over previous
import functools

import jax
import jax.numpy as jnp
from jax import lax
from jax.experimental import pallas as pl
from jax.experimental.pallas import tpu as pltpu
from jax.experimental.pallas import tpu_sc as plsc

BATCH = 16384
EMB = 32
HID = 16
PACK = 4
LANES = 128

NUM_CORES = 2
NUM_SUBCORES = 16
NUM_WORKERS = NUM_CORES * NUM_SUBCORES
ROWS_PER_WORKER = BATCH // NUM_WORKERS
CHUNK = 128
N_CHUNKS = ROWS_PER_WORKER // CHUNK
GROUPS = CHUNK // 16
NBUF = 2


def _sc_gather_mul(x0_hbm, x1_hbm, ptab_hbm, itab_hbm, h_hbm,
                   x0_v, x1_v, idx0_v, idx1_v, off_v,
                   prows_v, irows_v, h_v, sem0, sem1):
  wid = lax.axis_index("s") * NUM_CORES + lax.axis_index("c")
  base = wid * ROWS_PER_WORKER

  pltpu.sync_copy(x0_hbm.at[pl.ds(wid * N_CHUNKS, N_CHUNKS)], x0_v)
  pltpu.sync_copy(x1_hbm.at[pl.ds(wid * N_CHUNKS, N_CHUNKS)], x1_v)

  for j in range(N_CHUNKS):
    for g in range(GROUPS):
      sl = pl.ds(g * 16, 16)
      idx0_v[j, sl] = lax.shift_right_logical(x0_v[j, sl], 2)
      idx1_v[j, sl] = lax.shift_right_logical(x1_v[j, sl], 2)
      off_v[0, j, sl] = (x0_v[j, sl] & 3) * EMB
      off_v[1, j, sl] = (x1_v[j, sl] & 3) * EMB

  def fire(j, buf):
    cp = pltpu.make_async_copy(
        ptab_hbm.at[idx0_v.at[j]], prows_v.at[buf], sem0)
    ci = pltpu.make_async_copy(
        itab_hbm.at[idx1_v.at[j]], irows_v.at[buf], sem1)
    cp.start()
    ci.start()
    return cp, ci

  def drain(buf):
    pltpu.make_async_copy(
        ptab_hbm.at[idx0_v.at[0]], prows_v.at[buf], sem0).wait()
    pltpu.make_async_copy(
        itab_hbm.at[idx1_v.at[0]], irows_v.at[buf], sem1).wait()

  fire(0, 0)

  lane = lax.iota(jnp.int32, 16)

  def extract(j, buf):
    def body(g, _):
      off0g = off_v[0, j, pl.ds(g * 16, 16)]
      off1g = off_v[1, j, pl.ds(g * 16, 16)]
      for rr in range(16):
        sel = jnp.full((16,), rr, jnp.int32)
        col0 = off0g[sel] + lane
        col1 = off1g[sel] + lane
        row = jnp.full((16,), g * 16 + rr, jnp.int32)
        p_lo = plsc.load_gather(prows_v.at[buf], [row, col0])
        p_hi = plsc.load_gather(prows_v.at[buf], [row, col0 + 16])
        i_lo = plsc.load_gather(irows_v.at[buf], [row, col1])
        i_hi = plsc.load_gather(irows_v.at[buf], [row, col1 + 16])
        q = j * (CHUNK // PACK) + g * PACK + rr // PACK
        d = (rr % PACK) * EMB
        h_v[q, pl.ds(d, 16)] = p_lo * i_lo
        h_v[q, pl.ds(d + 16, 16)] = p_hi * i_hi
      return _
    lax.fori_loop(0, GROUPS, body, 0)

  for j in range(N_CHUNKS):
    if j + 1 < N_CHUNKS:
      fire(j + 1, (j + 1) % NBUF)
    drain(j % NBUF)
    extract(j, j % NBUF)

  pltpu.sync_copy(h_v, h_hbm.at[pl.ds(wid * (ROWS_PER_WORKER // PACK),
                                      ROWS_PER_WORKER // PACK)])


def _tc_bn_mlp(h_ref, gamma_ref, beta_ref, w1_ref, b1_ref, w2_ref, b2_ref,
               out_ref):
  h = h_ref[...]
  s = jnp.sum(h, axis=0, keepdims=True)
  sq = jnp.sum(h * h, axis=0, keepdims=True)
  s32 = (s[:, 0:32] + s[:, 32:64]) + (s[:, 64:96] + s[:, 96:128])
  sq32 = (sq[:, 0:32] + sq[:, 32:64]) + (sq[:, 64:96] + sq[:, 96:128])
  mean = s32 * (1.0 / BATCH)
  var = sq32 * (1.0 / BATCH) - mean * mean
  a32 = gamma_ref[...] * lax.rsqrt(var + 1e-5)
  c32 = beta_ref[...] - mean * a32
  a = jnp.concatenate([a32, a32, a32, a32], axis=1)
  c = jnp.concatenate([c32, c32, c32, c32], axis=1)
  hn = h * a + c

  z1 = jnp.dot(hn, w1_ref[...], preferred_element_type=jnp.float32)
  z1 = z1 + b1_ref[...]
  scale = 1.0507009873554804934193349852946
  alpha = 1.6732632423543772848170429916717
  z1 = scale * jnp.where(z1 > 0, z1, alpha * (jnp.exp(z1) - 1.0))
  z2 = jnp.dot(z1, w2_ref[...], preferred_element_type=jnp.float32)
  z2 = z2 + b2_ref[...]
  out_ref[...] = jax.nn.sigmoid(z2)


USED_ROWS = 40000
PAD_COLS = 40064
PACKED_ROWS = PAD_COLS // PACK
PCOLS = 1280
PTAIL = PAD_COLS - 31 * PCOLS


def _sc_pack(pt_hbm, it_hbm, ptab_hbm, itab_hbm, in_v, out_v):
  wid = lax.axis_index("s") * NUM_CORES + lax.axis_index("c")
  lane = lax.iota(jnp.int32, 16)
  rots = [(lane + k) & 15 for k in range(16)]
  rsh = [(rots[k] << 5) + lane for k in range(16)]

  def pack_block(src_hbm, dst_hbm, c0, cols):
    c0 = pl.multiple_of(c0, 128)
    pltpu.sync_copy(src_hbm.at[:, pl.ds(c0, cols)],
                    in_v.at[:, pl.ds(0, cols)])

    def body(cb, _):
      c_base = cb * 16
      cbv = jnp.full((16,), c_base, jnp.int32)
      for f_base in range(0, EMB, 16):
        fl = lane + f_base
        csf = jnp.full((16,), c_base * 32 + f_base, jnp.int32)
        for k in range(16):
          v = plsc.load_gather(in_v, [fl, rots[k] + cbv])
          flat = rsh[k] + csf
          plsc.store_scatter(out_v, [lax.shift_right_logical(flat, 7),
                                     flat & 127], v)
      return _

    lax.fori_loop(0, cols // 16, body, 0, unroll=2)
    q0 = pl.multiple_of(c0 // PACK, 32)
    pltpu.sync_copy(out_v.at[pl.ds(0, cols // PACK)],
                    dst_hbm.at[pl.ds(q0, cols // PACK)])

  @pl.when(wid < 31)
  def _():
    pack_block(pt_hbm, ptab_hbm, wid * PCOLS, PCOLS)
    pack_block(it_hbm, itab_hbm, wid * PCOLS, PCOLS)

  @pl.when(wid == 31)
  def _():
    pack_block(pt_hbm, ptab_hbm, 31 * PCOLS, PTAIL)
    pack_block(it_hbm, itab_hbm, 31 * PCOLS, PTAIL)


@jax.jit
def kernel(x, playlist_emb, item_emb, bn_gamma, bn_beta, W1, b1, W2, b2):
  xt = x.T.astype(jnp.int32)
  x0 = xt[0].reshape(LANES, LANES)
  x1 = xt[1].reshape(LANES, LANES)

  mesh = plsc.VectorSubcoreMesh(core_axis_name="c", subcore_axis_name="s")
  pack = pl.kernel(
      _sc_pack,
      out_type=(jax.ShapeDtypeStruct((PACKED_ROWS, LANES), jnp.float32),
                jax.ShapeDtypeStruct((PACKED_ROWS, LANES), jnp.float32)),
      mesh=mesh,
      compiler_params=pltpu.CompilerParams(
          use_tc_tiling_on_sc=True, needs_layout_passes=False),
      scratch_types=[
          pltpu.VMEM((EMB, PCOLS), jnp.float32),
          pltpu.VMEM((PCOLS // PACK, LANES), jnp.float32),
      ],
  )
  itp = jnp.pad(item_emb.T, ((0, 0), (0, PAD_COLS - USED_ROWS)))
  ptab, itab = pack(playlist_emb.T, itp)

  mesh = plsc.VectorSubcoreMesh(core_axis_name="c", subcore_axis_name="s")
  gather_mul = pl.kernel(
      _sc_gather_mul,
      out_type=jax.ShapeDtypeStruct((BATCH // PACK, PACK * EMB), jnp.float32),
      mesh=mesh,
      compiler_params=pltpu.CompilerParams(
          use_tc_tiling_on_sc=True, needs_layout_passes=False),
      scratch_types=[
          pltpu.VMEM((N_CHUNKS, CHUNK), jnp.int32),
          pltpu.VMEM((N_CHUNKS, CHUNK), jnp.int32),
          pltpu.VMEM((N_CHUNKS, CHUNK), jnp.int32),
          pltpu.VMEM((N_CHUNKS, CHUNK), jnp.int32),
          pltpu.VMEM((2, N_CHUNKS, CHUNK), jnp.int32),
          pltpu.VMEM((NBUF, CHUNK, LANES), jnp.float32),
          pltpu.VMEM((NBUF, CHUNK, LANES), jnp.float32),
          pltpu.VMEM((ROWS_PER_WORKER // PACK, PACK * EMB), jnp.float32),
          pltpu.SemaphoreType.DMA,
          pltpu.SemaphoreType.DMA,
      ],
  )
  h2 = gather_mul(x0, x1, ptab, itab)

  eye = jnp.eye(PACK, dtype=jnp.float32)
  w1big = jnp.kron(eye, W1.T)
  b1big = jnp.tile(b1, PACK).reshape(1, PACK * HID)
  w2big = jnp.kron(eye, W2.T)
  b2big = jnp.tile(b2, PACK).reshape(1, PACK)

  out = pl.pallas_call(
      _tc_bn_mlp,
      out_shape=jax.ShapeDtypeStruct((BATCH // PACK, PACK), jnp.float32),
  )(h2, bn_gamma.reshape(1, EMB), bn_beta.reshape(1, EMB),
    w1big, b1big, w2big, b2big)

  return out.reshape(BATCH, 1)
